# Initial kernel scaffold; baseline (speedup 1.0000x reference)
import jax
import jax.numpy as jnp
from jax.experimental import pallas as pl


def kernel(edge_index, x, edge_label_index, gcn_W, gcn_b, sg1_W, sg1_b, sg2_W, sg2_b, bn_gamma, bn_beta, bn_mean, bn_var, w_ih_f, w_hh_f, b_ih_f, b_hh_f, w_ih_b, w_hh_b, b_ih_b, b_hh_b, att_W, att_b, pred_W1, pred_b1, pbn_gamma, pbn_beta, pbn_mean, pbn_var, pred_W2, pred_b2):
    raise NotImplementedError("write your pallas kernel here")


# speedup vs baseline: 7.2204x; 7.2204x over previous
import functools

import jax
import jax.numpy as jnp
from jax import lax
from jax.experimental import pallas as pl
from jax.experimental.pallas import tpu as pltpu
from jax.experimental.pallas import tpu_sc as plsc

_N = 10000
_E = 320000
_H = 128
_NC = 2
_NS = 16
_NW = _NC * _NS
_EPW = _E // _NW
_CHUNK = 128
_NCHUNK = 80
_PAD_PW = _NCHUNK * _CHUNK - _EPW
_NP = 10240
_WB = _NP // _NS

_BLK = 256
_GRID_N = (_N + _BLK - 1) // _BLK

_NL = 32768
_LPW = _NL // _NW
_LCH = _LPW // _CHUNK

_LH = 192
_LHP = 256
_GP = 4 * _LHP

_sc_mesh = plsc.VectorSubcoreMesh(
    core_axis_name="c", subcore_axis_name="s", num_cores=_NC, num_subcores=_NS)


@functools.partial(
    pl.kernel,
    out_type=jax.ShapeDtypeStruct((_NC, _NP, 16), jnp.float32),
    mesh=_sc_mesh,
    scratch_types=[
        pltpu.VMEM_SHARED((_NP, 16), jnp.float32),
        pltpu.VMEM((_NCHUNK, _CHUNK), jnp.int32),
        pltpu.VMEM((_CHUNK, 16), jnp.float32),
    ],
)
def _sc_deg(dst_hbm, zeros_hbm, ones_hbm, out_hbm, acc_sh, dst_v, ones_v):
    cid = lax.axis_index("c")
    sid = lax.axis_index("s")
    wid = sid * _NC + cid
    pltpu.sync_copy(zeros_hbm, acc_sh.at[pl.ds(sid * _WB, _WB)])
    pltpu.sync_copy(dst_hbm.at[wid], dst_v)
    pltpu.sync_copy(ones_hbm, ones_v)
    plsc.subcore_barrier()

    def step(j, carry):
        pltpu.sync_copy(ones_v, acc_sh.at[dst_v.at[j]], add=True)
        return carry

    lax.fori_loop(0, _NCHUNK, step, 0)
    plsc.subcore_barrier()
    pltpu.sync_copy(acc_sh.at[pl.ds(sid * _WB, _WB)],
                    out_hbm.at[cid, pl.ds(sid * _WB, _WB)])


@functools.partial(
    pl.kernel,
    out_type=jax.ShapeDtypeStruct((_NC, _NP, _H), jnp.float32),
    mesh=_sc_mesh,
    scratch_types=[
        pltpu.VMEM_SHARED((_NP, _H), jnp.float32),
        pltpu.VMEM((_NCHUNK, _CHUNK), jnp.int32),
        pltpu.VMEM((_NCHUNK, _CHUNK), jnp.int32),
        pltpu.VMEM((_CHUNK, _H), jnp.float32),
        pltpu.SemaphoreType.DMA,
    ],
)
def _sc_prop(src_hbm, dst_hbm, feat_hbm, zeros_hbm, out_hbm,
             acc_sh, src_v, dst_v, rows_v, sem):
    cid = lax.axis_index("c")
    sid = lax.axis_index("s")
    wid = sid * _NC + cid
    pltpu.sync_copy(zeros_hbm, acc_sh.at[pl.ds(sid * _WB, _WB)])
    pltpu.sync_copy(src_hbm.at[wid], src_v)
    pltpu.sync_copy(dst_hbm.at[wid], dst_v)
    plsc.subcore_barrier()

    def step(j, carry):
        pltpu.async_copy(feat_hbm.at[src_v.at[j]], rows_v, sem).wait()
        pltpu.sync_copy(rows_v, acc_sh.at[dst_v.at[j]], add=True)
        return carry

    lax.fori_loop(0, _NCHUNK, step, 0)
    plsc.subcore_barrier()
    pltpu.sync_copy(acc_sh.at[pl.ds(sid * _WB, _WB)],
                    out_hbm.at[cid, pl.ds(sid * _WB, _WB)])


@functools.partial(
    pl.kernel,
    out_type=(jax.ShapeDtypeStruct((_NL, _H), jnp.float32),
              jax.ShapeDtypeStruct((_NL, _H), jnp.float32)),
    mesh=_sc_mesh,
    scratch_types=[
        pltpu.VMEM((_LCH, _CHUNK), jnp.int32),
        pltpu.VMEM((_LCH, _CHUNK), jnp.int32),
        pltpu.VMEM((_CHUNK, _H), jnp.float32),
        pltpu.VMEM((_CHUNK, _H), jnp.float32),
        pltpu.SemaphoreType.DMA,
        pltpu.SemaphoreType.DMA,
    ],
)
def _sc_lgather(row_hbm, col_hbm, gr_hbm, gc_hbm, zr_hbm, zc_hbm,
                ridx_v, cidx_v, rows_a, rows_b, sem_a, sem_b):
    cid = lax.axis_index("c")
    sid = lax.axis_index("s")
    wid = sid * _NC + cid
    base = wid * _LPW
    pltpu.sync_copy(row_hbm.at[wid], ridx_v)
    pltpu.sync_copy(col_hbm.at[wid], cidx_v)

    def step(j, carry):
        pltpu.async_copy(gr_hbm.at[ridx_v.at[j]], rows_a, sem_a).wait()
        pltpu.sync_copy(rows_a, zr_hbm.at[pl.ds(base + j * _CHUNK, _CHUNK)])
        pltpu.async_copy(gc_hbm.at[cidx_v.at[j]], rows_b, sem_b).wait()
        pltpu.sync_copy(rows_b, zc_hbm.at[pl.ds(base + j * _CHUNK, _CHUNK)])
        return carry

    lax.fori_loop(0, _LCH, step, 0)


def _dinv_of(degp):
    deg = degp[0, :, 0:1] + degp[1, :, 0:1] + 1.0
    return lax.rsqrt(deg)


def _bn(h, g, b, m, v):
    return (h - m) * lax.rsqrt(v + 1e-5) * g + b


def _ka_body(degp, x, w, xws_ref):
    dinv = _dinv_of(degp)
    xw = jnp.dot(x[...], w[...], preferred_element_type=jnp.float32)
    xws_ref[...] = xw * dinv


def _kb1_body(degp, p, xws, gcn_b, g, b, m, v, h0_ref, h0s_ref):
    dinv = _dinv_of(degp)
    s = (p[0] + p[1] + xws[...]) * dinv + gcn_b[...]
    s = _bn(jnp.maximum(s, 0.0), g[...], b[...], m[...], v[...])
    h0_ref[...] = s
    h0s_ref[...] = s * dinv


def _kb2_body(degp, p, h0s, w, bias, g, b, m, v, h1_ref, h1s_ref):
    dinv = _dinv_of(degp)
    t = (p[0] + p[1] + h0s[...]) * dinv
    s = jnp.dot(t, w[...], preferred_element_type=jnp.float32) + bias[...]
    s = _bn(jnp.maximum(s, 0.0), g[...], b[...], m[...], v[...])
    h1_ref[...] = s
    h1s_ref[...] = s * dinv


def _lstm_cell(x, h, c, wi, wh, bias):
    gt = (jnp.dot(x, wi, preferred_element_type=jnp.float32)
          + jnp.dot(h, wh, preferred_element_type=jnp.float32) + bias)
    i = jax.nn.sigmoid(gt[:, 0:_LHP])
    f = jax.nn.sigmoid(gt[:, _LHP:2 * _LHP])
    g = jnp.tanh(gt[:, 2 * _LHP:3 * _LHP])
    o = jax.nn.sigmoid(gt[:, 3 * _LHP:4 * _LHP])
    c2 = f * c + i * g
    return o * jnp.tanh(c2), c2


def _kbc_body(degp, p, h1s, h0, h1, sgw, sgb,
              wi_f, wh_f, bf, wi_b, wh_b, bb, af, ab, attb,
              w1r, w1c, b1, gr_ref, gc_ref):
    dinv = _dinv_of(degp)
    t = (p[0] + p[1] + h1s[...]) * dinv
    h2 = jnp.dot(t, sgw[...], preferred_element_type=jnp.float32) + sgb[...]
    xs = (h0[...], h1[...], h2)

    z = jnp.zeros((xs[0].shape[0], _LHP), jnp.float32)
    hcur, ccur = z, z
    hf = []
    for t_ in range(3):
        hcur, ccur = _lstm_cell(xs[t_], hcur, ccur, wi_f[...], wh_f[...], bf[...])
        hf.append(hcur)
    hcur, ccur = z, z
    hb = [None, None, None]
    for t_ in (2, 1, 0):
        hcur, ccur = _lstm_cell(xs[t_], hcur, ccur, wi_b[...], wh_b[...], bb[...])
        hb[t_] = hcur

    a = [jnp.dot(hf[t_], af[...], preferred_element_type=jnp.float32)
         + jnp.dot(hb[t_], ab[...], preferred_element_type=jnp.float32)
         + attb[...] for t_ in range(3)]
    mx = jnp.maximum(jnp.maximum(a[0], a[1]), a[2])
    e = [jnp.exp(av - mx) for av in a]
    ssum = e[0] + e[1] + e[2]
    jk = (e[0] * xs[0] + e[1] * xs[1] + e[2] * xs[2]) / ssum
    gr_ref[...] = jnp.dot(jk, w1r[...], preferred_element_type=jnp.float32) + b1[...]
    gc_ref[...] = jnp.dot(jk, w1c[...], preferred_element_type=jnp.float32)


def _kd_body(zr, zc, g, b, m, v, w2, b2, out_ref):
    s = jnp.maximum(zr[...] + zc[...], 0.0)
    s = _bn(s, g[...], b[...], m[...], v[...])
    out_ref[...] = jax.nn.sigmoid(
        jnp.dot(s, w2[...], preferred_element_type=jnp.float32) + b2[...])


def _nblk(i):
    return (i, 0)


_SPEC_DEG = pl.BlockSpec((_NC, _BLK, 16), lambda i: (0, i, 0))
_SPEC_P = pl.BlockSpec((_NC, _BLK, _H), lambda i: (0, i, 0))
_SPEC_NH = pl.BlockSpec((_BLK, _H), _nblk)


def _full(shape):
    return pl.BlockSpec(shape, lambda i: tuple(0 for _ in shape))


def _pad_gates_ih(w):
    wt = jnp.transpose(w)
    parts = [jnp.pad(wt[:, _LH * k:_LH * (k + 1)], ((0, 0), (0, _LHP - _LH)))
             for k in range(4)]
    return jnp.concatenate(parts, axis=1)


def _pad_gates_hh(w):
    wt = jnp.pad(jnp.transpose(w), ((0, _LHP - _LH), (0, 0)))
    parts = [jnp.pad(wt[:, _LH * k:_LH * (k + 1)], ((0, 0), (0, _LHP - _LH)))
             for k in range(4)]
    return jnp.concatenate(parts, axis=1)


def _pad_gates_b(b):
    parts = [jnp.pad(b[_LH * k:_LH * (k + 1)], (0, _LHP - _LH))
             for k in range(4)]
    return jnp.concatenate(parts).reshape(1, _GP)


def kernel(edge_index, x, edge_label_index, gcn_W, gcn_b, sg1_W, sg1_b,
           sg2_W, sg2_b, bn_gamma, bn_beta, bn_mean, bn_var,
           w_ih_f, w_hh_f, b_ih_f, b_hh_f, w_ih_b, w_hh_b, b_ih_b, b_hh_b,
           att_W, att_b, pred_W1, pred_b1, pbn_gamma, pbn_beta, pbn_mean,
           pbn_var, pred_W2, pred_b2):
    f32 = jnp.float32
    src = edge_index[0].astype(jnp.int32).reshape(_NW, _EPW)
    dst = edge_index[1].astype(jnp.int32).reshape(_NW, _EPW)
    pad_src = jnp.zeros((_NW, _PAD_PW), jnp.int32)
    pad_dst = jnp.broadcast_to(
        _N + jnp.arange(_NW, dtype=jnp.int32)[:, None], (_NW, _PAD_PW))
    srcp = jnp.concatenate([src, pad_src], 1).reshape(_NW, _NCHUNK, _CHUNK)
    dstp = jnp.concatenate([dst, pad_dst], 1).reshape(_NW, _NCHUNK, _CHUNK)

    zeros16 = jnp.zeros((_WB, 16), f32)
    zerosH = jnp.zeros((_WB, _H), f32)
    ones16 = jnp.ones((_CHUNK, 16), f32)

    degp = _sc_deg(dstp, zeros16, ones16)

    bnp = (bn_gamma.reshape(1, _H), bn_beta.reshape(1, _H),
           bn_mean.reshape(1, _H), bn_var.reshape(1, _H))

    xws = pl.pallas_call(
        _ka_body,
        grid=(_GRID_N,),
        in_specs=[_SPEC_DEG, _SPEC_NH, _full((_H, _H))],
        out_specs=_SPEC_NH,
        out_shape=jax.ShapeDtypeStruct((_N, _H), f32),
    )(degp, x, gcn_W)

    p1 = _sc_prop(srcp, dstp, xws, zerosH)

    h0, h0s = pl.pallas_call(
        _kb1_body,
        grid=(_GRID_N,),
        in_specs=[_SPEC_DEG, _SPEC_P, _SPEC_NH] + [_full((1, _H))] * 5,
        out_specs=(_SPEC_NH, _SPEC_NH),
        out_shape=(jax.ShapeDtypeStruct((_N, _H), f32),
                   jax.ShapeDtypeStruct((_N, _H), f32)),
    )(degp, p1, xws, gcn_b.reshape(1, _H), *bnp)

    p2 = _sc_prop(srcp, dstp, h0s, zerosH)

    h1, h1s = pl.pallas_call(
        _kb2_body,
        grid=(_GRID_N,),
        in_specs=[_SPEC_DEG, _SPEC_P, _SPEC_NH, _full((_H, _H))]
        + [_full((1, _H))] * 5,
        out_specs=(_SPEC_NH, _SPEC_NH),
        out_shape=(jax.ShapeDtypeStruct((_N, _H), f32),
                   jax.ShapeDtypeStruct((_N, _H), f32)),
    )(degp, p2, h0s, sg1_W, sg1_b.reshape(1, _H), *bnp)

    p3 = _sc_prop(srcp, dstp, h1s, zerosH)

    wi_f = _pad_gates_ih(w_ih_f)
    wh_f = _pad_gates_hh(w_hh_f)
    bf = _pad_gates_b(b_ih_f + b_hh_f)
    wi_b = _pad_gates_ih(w_ih_b)
    wh_b = _pad_gates_hh(w_hh_b)
    bb = _pad_gates_b(b_ih_b + b_hh_b)
    af = jnp.pad(att_W[:_LH], ((0, _LHP - _LH), (0, 0)))
    ab = jnp.pad(att_W[_LH:], ((0, _LHP - _LH), (0, 0)))
    w1r = pred_W1[:_H]
    w1c = pred_W1[_H:]

    gr, gc = pl.pallas_call(
        _kbc_body,
        grid=(_GRID_N,),
        in_specs=[_SPEC_DEG, _SPEC_P, _SPEC_NH, _SPEC_NH, _SPEC_NH,
                  _full((_H, _H)), _full((1, _H)),
                  _full((_H, _GP)), _full((_LHP, _GP)), _full((1, _GP)),
                  _full((_H, _GP)), _full((_LHP, _GP)), _full((1, _GP)),
                  _full((_LHP, 1)), _full((_LHP, 1)), _full((1, 1)),
                  _full((_H, _H)), _full((_H, _H)), _full((1, _H))],
        out_specs=(_SPEC_NH, _SPEC_NH),
        out_shape=(jax.ShapeDtypeStruct((_N, _H), f32),
                   jax.ShapeDtypeStruct((_N, _H), f32)),
    )(degp, p3, h1s, h0, h1, sg2_W, sg2_b.reshape(1, _H),
      wi_f, wh_f, bf, wi_b, wh_b, bb, af, ab, att_b.reshape(1, 1),
      w1r, w1c, pred_b1.reshape(1, _H))

    rowi = edge_label_index[0].astype(jnp.int32).reshape(_NW, _LCH, _CHUNK)
    coli = edge_label_index[1].astype(jnp.int32).reshape(_NW, _LCH, _CHUNK)
    zr, zc = _sc_lgather(rowi, coli, gr, gc)

    _DBLK = 1024
    pbnp = (pbn_gamma.reshape(1, _H), pbn_beta.reshape(1, _H),
            pbn_mean.reshape(1, _H), pbn_var.reshape(1, _H))
    out = pl.pallas_call(
        _kd_body,
        grid=(_NL // _DBLK,),
        in_specs=[pl.BlockSpec((_DBLK, _H), _nblk),
                  pl.BlockSpec((_DBLK, _H), _nblk)]
        + [_full((1, _H))] * 4 + [_full((_H, 1)), _full((1, 1))],
        out_specs=pl.BlockSpec((_DBLK, 1), _nblk),
        out_shape=jax.ShapeDtypeStruct((_NL, 1), f32),
    )(zr, zc, *pbnp, pred_W2, pred_b2.reshape(1, 1))
    return out


# speedup vs baseline: 8.0712x; 1.1178x over previous
import functools

import jax
import jax.numpy as jnp
from jax import lax
from jax.experimental import pallas as pl
from jax.experimental.pallas import tpu as pltpu
from jax.experimental.pallas import tpu_sc as plsc

_N = 10000
_E = 320000
_H = 128
_NC = 2
_NS = 16
_NW = _NC * _NS
_EPW = _E // _NW
_CHUNK = 128
_PCH = 128
_PNCH = 80
_PPH = 40
_PAD_PW = _PNCH * _PCH - _EPW
_NP = 10240
_WB = _NP // _NS

_BLK = 256
_GRID_N = (_N + _BLK - 1) // _BLK

_NL = 32768
_LPW = _NL // _NW
_LCH = _LPW // _CHUNK

_LH = 192
_LHP = 256
_GP = 4 * _LHP

_sc_mesh = plsc.VectorSubcoreMesh(
    core_axis_name="c", subcore_axis_name="s", num_cores=_NC, num_subcores=_NS)


@functools.partial(
    pl.kernel,
    out_type=jax.ShapeDtypeStruct((_NC, _NP, 16), jnp.float32),
    mesh=_sc_mesh,
    scratch_types=[
        pltpu.VMEM_SHARED((_NP, 16), jnp.float32),
        pltpu.VMEM((_PNCH, _PCH), jnp.int32),
        pltpu.VMEM((_PCH, 16), jnp.float32),

    ],
)
def _sc_deg(dst_hbm, zeros_hbm, ones_hbm, out_hbm, acc_sh, dst_v, ones_v):
    cid = lax.axis_index("c")
    sid = lax.axis_index("s")
    wid = sid * _NC + cid
    pltpu.sync_copy(zeros_hbm, acc_sh.at[pl.ds(sid * _WB, _WB)])
    pltpu.sync_copy(dst_hbm.at[wid], dst_v)
    pltpu.sync_copy(ones_hbm, ones_v)
    plsc.subcore_barrier()

    def step(j, carry):
        pltpu.sync_copy(ones_v, acc_sh.at[dst_v.at[j]], add=True)
        return carry

    lax.fori_loop(0, _PNCH, step, 0)
    plsc.subcore_barrier()
    pltpu.sync_copy(acc_sh.at[pl.ds(sid * _WB, _WB)],
                    out_hbm.at[cid, pl.ds(sid * _WB, _WB)])


@functools.partial(
    pl.kernel,
    out_type=jax.ShapeDtypeStruct((_NC, _NP, _H), jnp.float32),
    mesh=_sc_mesh,
    scratch_types=[
        pltpu.VMEM_SHARED((_NP, _H), jnp.float32),
        pltpu.VMEM((_PPH, _PCH), jnp.int32),
        pltpu.VMEM((_PPH, _PCH), jnp.int32),
        pltpu.VMEM((_PCH, _H), jnp.float32),
        pltpu.VMEM((_PCH, _H), jnp.float32),
        pltpu.SemaphoreType.DMA,
        pltpu.SemaphoreType.DMA,
    ],
)
def _sc_prop(src_hbm, dst_hbm, feat_hbm, zeros_hbm, out_hbm,
             acc_sh, src_v, dst_v, rows_a, rows_b, sem_a, sem_b):
    cid = lax.axis_index("c")
    sid = lax.axis_index("s")
    wid = sid * _NC + cid
    pltpu.sync_copy(zeros_hbm, acc_sh.at[pl.ds(sid * _WB, _WB)])
    plsc.subcore_barrier()

    for ph in range(_PNCH // _PPH):
        pltpu.sync_copy(src_hbm.at[wid, pl.ds(ph * _PPH, _PPH)], src_v)
        pltpu.sync_copy(dst_hbm.at[wid, pl.ds(ph * _PPH, _PPH)], dst_v)
        pltpu.async_copy(feat_hbm.at[src_v.at[0]], rows_a, sem_a)

        def step(jh, carry):
            j = jh * 2
            pltpu.async_copy(feat_hbm.at[src_v.at[j + 1]], rows_b, sem_b)
            pltpu.make_async_copy(feat_hbm.at[src_v.at[j]], rows_a,
                                  sem_a).wait()
            pltpu.sync_copy(rows_a, acc_sh.at[dst_v.at[j]], add=True)

            @pl.when(j + 2 < _PPH)
            def _():
                pltpu.async_copy(feat_hbm.at[src_v.at[j + 2]], rows_a, sem_a)

            pltpu.make_async_copy(feat_hbm.at[src_v.at[j + 1]], rows_b,
                                  sem_b).wait()
            pltpu.sync_copy(rows_b, acc_sh.at[dst_v.at[j + 1]], add=True)
            return carry

        lax.fori_loop(0, _PPH // 2, step, 0)
    plsc.subcore_barrier()
    pltpu.sync_copy(acc_sh.at[pl.ds(sid * _WB, _WB)],
                    out_hbm.at[cid, pl.ds(sid * _WB, _WB)])


@functools.partial(
    pl.kernel,
    out_type=(jax.ShapeDtypeStruct((_NL, _H), jnp.float32),
              jax.ShapeDtypeStruct((_NL, _H), jnp.float32)),
    mesh=_sc_mesh,
    scratch_types=[
        pltpu.VMEM((_LCH, _CHUNK), jnp.int32),
        pltpu.VMEM((_LCH, _CHUNK), jnp.int32),
        pltpu.VMEM((_CHUNK, _H), jnp.float32),
        pltpu.VMEM((_CHUNK, _H), jnp.float32),
        pltpu.SemaphoreType.DMA,
        pltpu.SemaphoreType.DMA,
    ],
)
def _sc_lgather(row_hbm, col_hbm, gr_hbm, gc_hbm, zr_hbm, zc_hbm,
                ridx_v, cidx_v, rows_a, rows_b, sem_a, sem_b):
    cid = lax.axis_index("c")
    sid = lax.axis_index("s")
    wid = sid * _NC + cid
    base = wid * _LPW
    pltpu.sync_copy(row_hbm.at[wid], ridx_v)
    pltpu.sync_copy(col_hbm.at[wid], cidx_v)

    def step(j, carry):
        pltpu.async_copy(gr_hbm.at[ridx_v.at[j]], rows_a, sem_a).wait()
        pltpu.sync_copy(rows_a, zr_hbm.at[pl.ds(base + j * _CHUNK, _CHUNK)])
        pltpu.async_copy(gc_hbm.at[cidx_v.at[j]], rows_b, sem_b).wait()
        pltpu.sync_copy(rows_b, zc_hbm.at[pl.ds(base + j * _CHUNK, _CHUNK)])
        return carry

    lax.fori_loop(0, _LCH, step, 0)


def _dinv_of(degp):
    deg = degp[0, :, 0:1] + degp[1, :, 0:1] + 1.0
    return lax.rsqrt(deg)


def _bn(h, g, b, m, v):
    return (h - m) * lax.rsqrt(v + 1e-5) * g + b


def _ka_body(degp, x, w, xws_ref):
    dinv = _dinv_of(degp)
    xw = jnp.dot(x[...], w[...], preferred_element_type=jnp.float32)
    xws_ref[...] = xw * dinv


def _kb1_body(degp, p, xws, gcn_b, g, b, m, v, h0_ref, h0s_ref):
    dinv = _dinv_of(degp)
    s = (p[0] + p[1] + xws[...]) * dinv + gcn_b[...]
    s = _bn(jnp.maximum(s, 0.0), g[...], b[...], m[...], v[...])
    h0_ref[...] = s
    h0s_ref[...] = s * dinv


def _kb2_body(degp, p, h0s, w, bias, g, b, m, v, h1_ref, h1s_ref):
    dinv = _dinv_of(degp)
    t = (p[0] + p[1] + h0s[...]) * dinv
    s = jnp.dot(t, w[...], preferred_element_type=jnp.float32) + bias[...]
    s = _bn(jnp.maximum(s, 0.0), g[...], b[...], m[...], v[...])
    h1_ref[...] = s
    h1s_ref[...] = s * dinv


def _lstm_cell(x, h, c, wi, wh, bias):
    gt = (jnp.dot(x, wi, preferred_element_type=jnp.float32)
          + jnp.dot(h, wh, preferred_element_type=jnp.float32) + bias)
    i = jax.nn.sigmoid(gt[:, 0:_LHP])
    f = jax.nn.sigmoid(gt[:, _LHP:2 * _LHP])
    g = jnp.tanh(gt[:, 2 * _LHP:3 * _LHP])
    o = jax.nn.sigmoid(gt[:, 3 * _LHP:4 * _LHP])
    c2 = f * c + i * g
    return o * jnp.tanh(c2), c2


def _kbc_body(degp, p, h1s, h0, h1, sgw, sgb,
              wi_f, wh_f, bf, wi_b, wh_b, bb, af, ab, attb,
              w1r, w1c, b1, gr_ref, gc_ref):
    dinv = _dinv_of(degp)
    t = (p[0] + p[1] + h1s[...]) * dinv
    h2 = jnp.dot(t, sgw[...], preferred_element_type=jnp.float32) + sgb[...]
    xs = (h0[...], h1[...], h2)

    z = jnp.zeros((xs[0].shape[0], _LHP), jnp.float32)
    hcur, ccur = z, z
    hf = []
    for t_ in range(3):
        hcur, ccur = _lstm_cell(xs[t_], hcur, ccur, wi_f[...], wh_f[...], bf[...])
        hf.append(hcur)
    hcur, ccur = z, z
    hb = [None, None, None]
    for t_ in (2, 1, 0):
        hcur, ccur = _lstm_cell(xs[t_], hcur, ccur, wi_b[...], wh_b[...], bb[...])
        hb[t_] = hcur

    a = [jnp.dot(hf[t_], af[...], preferred_element_type=jnp.float32)
         + jnp.dot(hb[t_], ab[...], preferred_element_type=jnp.float32)
         + attb[...] for t_ in range(3)]
    mx = jnp.maximum(jnp.maximum(a[0], a[1]), a[2])
    e = [jnp.exp(av - mx) for av in a]
    ssum = e[0] + e[1] + e[2]
    jk = (e[0] * xs[0] + e[1] * xs[1] + e[2] * xs[2]) / ssum
    gr_ref[...] = jnp.dot(jk, w1r[...], preferred_element_type=jnp.float32) + b1[...]
    gc_ref[...] = jnp.dot(jk, w1c[...], preferred_element_type=jnp.float32)


def _kd_body(zr, zc, g, b, m, v, w2, b2, out_ref):
    s = jnp.maximum(zr[...] + zc[...], 0.0)
    s = _bn(s, g[...], b[...], m[...], v[...])
    out_ref[...] = jax.nn.sigmoid(
        jnp.dot(s, w2[...], preferred_element_type=jnp.float32) + b2[...])


def _nblk(i):
    return (i, 0)


_SPEC_DEG = pl.BlockSpec((_NC, _BLK, 16), lambda i: (0, i, 0))
_SPEC_P = pl.BlockSpec((_NC, _BLK, _H), lambda i: (0, i, 0))
_SPEC_NH = pl.BlockSpec((_BLK, _H), _nblk)


def _full(shape):
    return pl.BlockSpec(shape, lambda i: tuple(0 for _ in shape))


def _pad_gates_ih(w):
    wt = jnp.transpose(w)
    parts = [jnp.pad(wt[:, _LH * k:_LH * (k + 1)], ((0, 0), (0, _LHP - _LH)))
             for k in range(4)]
    return jnp.concatenate(parts, axis=1)


def _pad_gates_hh(w):
    wt = jnp.pad(jnp.transpose(w), ((0, _LHP - _LH), (0, 0)))
    parts = [jnp.pad(wt[:, _LH * k:_LH * (k + 1)], ((0, 0), (0, _LHP - _LH)))
             for k in range(4)]
    return jnp.concatenate(parts, axis=1)


def _pad_gates_b(b):
    parts = [jnp.pad(b[_LH * k:_LH * (k + 1)], (0, _LHP - _LH))
             for k in range(4)]
    return jnp.concatenate(parts).reshape(1, _GP)


def kernel(edge_index, x, edge_label_index, gcn_W, gcn_b, sg1_W, sg1_b,
           sg2_W, sg2_b, bn_gamma, bn_beta, bn_mean, bn_var,
           w_ih_f, w_hh_f, b_ih_f, b_hh_f, w_ih_b, w_hh_b, b_ih_b, b_hh_b,
           att_W, att_b, pred_W1, pred_b1, pbn_gamma, pbn_beta, pbn_mean,
           pbn_var, pred_W2, pred_b2):
    f32 = jnp.float32
    src = edge_index[0].astype(jnp.int32).reshape(_NW, _EPW)
    dst = edge_index[1].astype(jnp.int32).reshape(_NW, _EPW)
    pad_src = jnp.zeros((_NW, _PAD_PW), jnp.int32)
    pad_dst = jnp.broadcast_to(
        _N + jnp.arange(_NW, dtype=jnp.int32)[:, None], (_NW, _PAD_PW))
    srcp = jnp.concatenate([src, pad_src], 1).reshape(_NW, _PNCH, _PCH)
    dstp = jnp.concatenate([dst, pad_dst], 1).reshape(_NW, _PNCH, _PCH)

    zeros16 = jnp.zeros((_WB, 16), f32)
    zerosH = jnp.zeros((_WB, _H), f32)
    ones16 = jnp.ones((_PCH, 16), f32)

    degp = _sc_deg(dstp, zeros16, ones16)

    bnp = (bn_gamma.reshape(1, _H), bn_beta.reshape(1, _H),
           bn_mean.reshape(1, _H), bn_var.reshape(1, _H))

    xws = pl.pallas_call(
        _ka_body,
        grid=(_GRID_N,),
        in_specs=[_SPEC_DEG, _SPEC_NH, _full((_H, _H))],
        out_specs=_SPEC_NH,
        out_shape=jax.ShapeDtypeStruct((_N, _H), f32),
    )(degp, x, gcn_W)

    p1 = _sc_prop(srcp, dstp, xws, zerosH)

    h0, h0s = pl.pallas_call(
        _kb1_body,
        grid=(_GRID_N,),
        in_specs=[_SPEC_DEG, _SPEC_P, _SPEC_NH] + [_full((1, _H))] * 5,
        out_specs=(_SPEC_NH, _SPEC_NH),
        out_shape=(jax.ShapeDtypeStruct((_N, _H), f32),
                   jax.ShapeDtypeStruct((_N, _H), f32)),
    )(degp, p1, xws, gcn_b.reshape(1, _H), *bnp)

    p2 = _sc_prop(srcp, dstp, h0s, zerosH)

    h1, h1s = pl.pallas_call(
        _kb2_body,
        grid=(_GRID_N,),
        in_specs=[_SPEC_DEG, _SPEC_P, _SPEC_NH, _full((_H, _H))]
        + [_full((1, _H))] * 5,
        out_specs=(_SPEC_NH, _SPEC_NH),
        out_shape=(jax.ShapeDtypeStruct((_N, _H), f32),
                   jax.ShapeDtypeStruct((_N, _H), f32)),
    )(degp, p2, h0s, sg1_W, sg1_b.reshape(1, _H), *bnp)

    p3 = _sc_prop(srcp, dstp, h1s, zerosH)

    wi_f = _pad_gates_ih(w_ih_f)
    wh_f = _pad_gates_hh(w_hh_f)
    bf = _pad_gates_b(b_ih_f + b_hh_f)
    wi_b = _pad_gates_ih(w_ih_b)
    wh_b = _pad_gates_hh(w_hh_b)
    bb = _pad_gates_b(b_ih_b + b_hh_b)
    af = jnp.pad(att_W[:_LH], ((0, _LHP - _LH), (0, 0)))
    ab = jnp.pad(att_W[_LH:], ((0, _LHP - _LH), (0, 0)))
    w1r = pred_W1[:_H]
    w1c = pred_W1[_H:]

    gr, gc = pl.pallas_call(
        _kbc_body,
        grid=(_GRID_N,),
        in_specs=[_SPEC_DEG, _SPEC_P, _SPEC_NH, _SPEC_NH, _SPEC_NH,
                  _full((_H, _H)), _full((1, _H)),
                  _full((_H, _GP)), _full((_LHP, _GP)), _full((1, _GP)),
                  _full((_H, _GP)), _full((_LHP, _GP)), _full((1, _GP)),
                  _full((_LHP, 1)), _full((_LHP, 1)), _full((1, 1)),
                  _full((_H, _H)), _full((_H, _H)), _full((1, _H))],
        out_specs=(_SPEC_NH, _SPEC_NH),
        out_shape=(jax.ShapeDtypeStruct((_N, _H), f32),
                   jax.ShapeDtypeStruct((_N, _H), f32)),
    )(degp, p3, h1s, h0, h1, sg2_W, sg2_b.reshape(1, _H),
      wi_f, wh_f, bf, wi_b, wh_b, bb, af, ab, att_b.reshape(1, 1),
      w1r, w1c, pred_b1.reshape(1, _H))

    rowi = edge_label_index[0].astype(jnp.int32).reshape(_NW, _LCH, _CHUNK)
    coli = edge_label_index[1].astype(jnp.int32).reshape(_NW, _LCH, _CHUNK)
    zr, zc = _sc_lgather(rowi, coli, gr, gc)

    _DBLK = 1024
    pbnp = (pbn_gamma.reshape(1, _H), pbn_beta.reshape(1, _H),
            pbn_mean.reshape(1, _H), pbn_var.reshape(1, _H))
    out = pl.pallas_call(
        _kd_body,
        grid=(_NL // _DBLK,),
        in_specs=[pl.BlockSpec((_DBLK, _H), _nblk),
                  pl.BlockSpec((_DBLK, _H), _nblk)]
        + [_full((1, _H))] * 4 + [_full((_H, 1)), _full((1, 1))],
        out_specs=pl.BlockSpec((_DBLK, 1), _nblk),
        out_shape=jax.ShapeDtypeStruct((_NL, 1), f32),
    )(zr, zc, *pbnp, pred_W2, pred_b2.reshape(1, 1))
    return out
